# baseline (speedup 1.0000x reference)
import jax
import jax.numpy as jnp
import numpy as np
from jax.experimental import pallas as pl
from jax.experimental.pallas import tpu as pltpu

_T = 0.2
_N = 50
_DIM = 2
_R = 1.0
_GAMMA = 1.0
_SIGMA = float(np.sqrt(2.0))
_DT = _T / _N

_LANES = 128
_ROWS = 128
_BS = 32


def _scan_kernel(u_ref, x_ref, dw_ref, y_ref):
    u = u_ref[0]
    x0c = x_ref[0]
    x1c = x_ref[1]
    zeros = jnp.zeros_like(x0c)
    ones = jnp.ones_like(x0c)
    e_step = jnp.exp(jnp.full_like(x0c, -_GAMMA * _DT) * 1.0)

    def step(i, carry):
        x0c, x1c, flag, y, disc, df0e, df1e, disce = carry
        d0 = dw_ref[i, 0]
        d1 = dw_ref[i, 1]
        drift0 = u * x0c * _DT
        drift1 = u * x1c * _DT
        diff0 = _SIGMA * d0
        diff1 = _SIGMA * d1
        xt0 = x0c + drift0 + diff0
        xt1 = x1c + drift1 + diff1
        bval = (xt0 ** 2 + xt1 ** 2) - _R ** 2
        active = flag > 0.0
        exit_now = active & (bval >= 0)
        cont = active & (bval < 0)
        w = (u ** 2 + 2.0) * (x0c ** 2 + x1c ** 2) - 2.0 * _DIM
        y = jnp.where(cont, y + w * _DT * disc, y)
        df0e = jnp.where(exit_now, diff0, df0e)
        df1e = jnp.where(exit_now, diff1, df1e)
        disce = jnp.where(exit_now, disc, disce)
        x0c = jnp.where(cont, xt0, x0c)
        x1c = jnp.where(cont, xt1, x1c)
        disc = disc * e_step
        flag = jnp.where(cont, 1.0, 0.0)
        return x0c, x1c, flag, y, disc, df0e, df1e, disce

    x0c, x1c, flag, y, disc, df0e, df1e, disce = jax.lax.fori_loop(
        0, _N, step,
        (x0c, x1c, ones, zeros, ones, zeros, zeros, ones), unroll=5)

    drift0 = u * x0c * _DT
    drift1 = u * x1c * _DT
    diff0 = df0e
    diff1 = df1e
    a = drift0 ** 2 + drift1 ** 2
    b = 2.0 * (drift0 * diff0 + drift1 * diff1)
    c = (2.0 * drift0 * x0c + diff0 ** 2) + (2.0 * drift1 * x1c + diff1 ** 2)
    d = 2.0 * (diff0 * x0c + diff1 * x1c)
    e = (x0c ** 2 + x1c ** 2) - _R ** 2
    p = (8 * a * c - 3 * b ** 2) / (8 * a ** 2)
    q = (b ** 3 - 4 * a * b * c + 8 * a ** 2 * d) / (8 * a ** 3)
    sign_q = jnp.sign(q)
    D0 = c ** 2 - 3 * b * d + 12 * a * e
    D1 = (2 * c ** 3 - 9 * b * c * d + 27 * b ** 2 * e
          + 27 * a * d ** 2 - 72 * a * c * e)
    D2 = D1 ** 2 - 4 * D0 ** 3
    sig2 = jnp.ceil((jnp.sign(D2) + 1.0) / 2.0)
    QQ = (D1 + jnp.sqrt(jnp.abs(D2))) / 2.0
    Q = jnp.sign(QQ) * jnp.exp(jnp.log(jnp.abs(QQ)) * (1.0 / 3.0))
    S_plus = 0.5 * jnp.sqrt(jnp.abs((Q + D0 / Q) / (3 * a) - 2 * p / 3))
    d0a = jnp.abs(D0)
    sqrt_d0a = jnp.sqrt(d0a)
    t_arg = jnp.clip(D1 / 2.0 / (d0a * sqrt_d0a), -1.0 + 1e-6, 1.0 - 1e-6)
    phi = 2.0 * jnp.arctan2(jnp.sqrt(1.0 - t_arg * t_arg), 1.0 + t_arg)
    S_minus = 0.5 * jnp.sqrt(jnp.abs(2 * sqrt_d0a * jnp.cos(phi / 3.0)
                                     / (3 * a) - 2 * p / 3))
    S = sig2 * S_plus + (1 - sig2) * S_minus
    temp = -4 * S ** 2 - 2 * p + jnp.abs(q / S)
    sqrt_rho = 0.5 * jnp.sqrt(jnp.abs(temp)) - b / (4 * a) - sign_q * S
    new_temp = -4 * S ** 2 - 2 * p - jnp.abs(q / S)
    new_sqrt_rho = 0.5 * jnp.sqrt(jnp.abs(new_temp)) - b / (4 * a) + sign_q * S
    srf = jnp.where((1 - sqrt_rho) * sqrt_rho < 0, new_sqrt_rho, sqrt_rho)
    rho = srf ** 2

    w_e = (u ** 2 + 2.0) * (x0c ** 2 + x1c ** 2) - 2.0 * _DIM
    pos = rho > 0
    csq = jnp.sqrt(jnp.where(pos, rho, 1.0)) * pos
    xn0 = x0c + drift0 * rho + diff0 * csq
    xn1 = x1c + drift1 * rho + diff1 * csq
    disc_n = disce * jnp.exp(-_GAMMA * _DT * rho)
    y_exit = (y + rho * w_e * _DT * disce) + disc_n * (xn0 ** 2 + xn1 ** 2)
    y_active = y + disc * (x0c ** 2 + x1c ** 2)
    y_ref[...] = jnp.where(flag > 0.0, y_active, y_exit)


def kernel(x0, dw, u):
    num_sample, dim = x0.shape
    xT = x0.T.reshape(dim, _ROWS, _LANES)
    d0T = dw[:, 0, :].T.reshape(_N, 1, _ROWS, _LANES)
    d1T = dw[:, 1, :].T.reshape(_N, 1, _ROWS, _LANES)
    dwT = jnp.concatenate([d0T, d1T], axis=1)
    u1 = jnp.reshape(u, (1,)).astype(jnp.float32)
    y = pl.pallas_call(
        _scan_kernel,
        grid=(_ROWS // _BS,),
        in_specs=[
            pl.BlockSpec(memory_space=pltpu.SMEM),
            pl.BlockSpec((dim, _BS, _LANES), lambda i: (0, i, 0)),
            pl.BlockSpec((_N, dim, _BS, _LANES), lambda i: (0, 0, i, 0)),
        ],
        out_specs=pl.BlockSpec((_BS, _LANES), lambda i: (i, 0)),
        out_shape=jax.ShapeDtypeStruct((_ROWS, _LANES), jnp.float32),
    )(u1, xT, dwT)
    return y.reshape(num_sample, 1)


# speedup vs baseline: 2.0760x; 2.0760x over previous
import jax
import jax.numpy as jnp
import numpy as np
from jax.experimental import pallas as pl
from jax.experimental.pallas import tpu as pltpu

_T = 0.2
_N = 50
_DIM = 2
_R = 1.0
_GAMMA = 1.0
_SIGMA = float(np.sqrt(2.0))
_DT = _T / _N

_LANES = 128
_ROWS = 128
_BS = 32


def _scan_kernel(u_ref, x_ref, dw_ref, y_ref):
    u = u_ref[0]
    x0c = x_ref[0]
    x1c = x_ref[1]
    zeros = jnp.zeros_like(x0c)
    ones = jnp.ones_like(x0c)
    e_step = jnp.exp(jnp.full_like(x0c, -_GAMMA * _DT) * 1.0)

    def step(i, carry):
        x0c, x1c, flag, y, disc, df0e, df1e, disce = carry
        d0 = dw_ref[i, 0]
        d1 = dw_ref[i, 1]
        drift0 = u * x0c * _DT
        drift1 = u * x1c * _DT
        diff0 = _SIGMA * d0
        diff1 = _SIGMA * d1
        xt0 = x0c + drift0 + diff0
        xt1 = x1c + drift1 + diff1
        bval = (xt0 ** 2 + xt1 ** 2) - _R ** 2
        active = flag > 0.0
        exit_now = active & (bval >= 0)
        cont = active & (bval < 0)
        w = (u ** 2 + 2.0) * (x0c ** 2 + x1c ** 2) - 2.0 * _DIM
        y = jnp.where(cont, y + w * _DT * disc, y)
        df0e = jnp.where(exit_now, diff0, df0e)
        df1e = jnp.where(exit_now, diff1, df1e)
        disce = jnp.where(exit_now, disc, disce)
        x0c = jnp.where(cont, xt0, x0c)
        x1c = jnp.where(cont, xt1, x1c)
        disc = disc * e_step
        flag = jnp.where(cont, 1.0, 0.0)
        return x0c, x1c, flag, y, disc, df0e, df1e, disce

    x0c, x1c, flag, y, disc, df0e, df1e, disce = jax.lax.fori_loop(
        0, _N, step,
        (x0c, x1c, ones, zeros, ones, zeros, zeros, ones), unroll=10)

    drift0 = u * x0c * _DT
    drift1 = u * x1c * _DT
    diff0 = df0e
    diff1 = df1e
    a = drift0 ** 2 + drift1 ** 2
    b = 2.0 * (drift0 * diff0 + drift1 * diff1)
    c = (2.0 * drift0 * x0c + diff0 ** 2) + (2.0 * drift1 * x1c + diff1 ** 2)
    d = 2.0 * (diff0 * x0c + diff1 * x1c)
    e = (x0c ** 2 + x1c ** 2) - _R ** 2
    p = (8 * a * c - 3 * b ** 2) / (8 * a ** 2)
    q = (b ** 3 - 4 * a * b * c + 8 * a ** 2 * d) / (8 * a ** 3)
    sign_q = jnp.sign(q)
    D0 = c ** 2 - 3 * b * d + 12 * a * e
    D1 = (2 * c ** 3 - 9 * b * c * d + 27 * b ** 2 * e
          + 27 * a * d ** 2 - 72 * a * c * e)
    D2 = D1 ** 2 - 4 * D0 ** 3
    sig2 = jnp.ceil((jnp.sign(D2) + 1.0) / 2.0)
    QQ = (D1 + jnp.sqrt(jnp.abs(D2))) / 2.0
    Q = jnp.sign(QQ) * jnp.exp(jnp.log(jnp.abs(QQ)) * (1.0 / 3.0))
    S_plus = 0.5 * jnp.sqrt(jnp.abs((Q + D0 / Q) / (3 * a) - 2 * p / 3))
    d0a = jnp.abs(D0)
    sqrt_d0a = jnp.sqrt(d0a)
    t_arg = jnp.clip(D1 / 2.0 / (d0a * sqrt_d0a), -1.0 + 1e-6, 1.0 - 1e-6)
    phi = 2.0 * jnp.arctan2(jnp.sqrt(1.0 - t_arg * t_arg), 1.0 + t_arg)
    S_minus = 0.5 * jnp.sqrt(jnp.abs(2 * sqrt_d0a * jnp.cos(phi / 3.0)
                                     / (3 * a) - 2 * p / 3))
    S = sig2 * S_plus + (1 - sig2) * S_minus
    temp = -4 * S ** 2 - 2 * p + jnp.abs(q / S)
    sqrt_rho = 0.5 * jnp.sqrt(jnp.abs(temp)) - b / (4 * a) - sign_q * S
    new_temp = -4 * S ** 2 - 2 * p - jnp.abs(q / S)
    new_sqrt_rho = 0.5 * jnp.sqrt(jnp.abs(new_temp)) - b / (4 * a) + sign_q * S
    srf = jnp.where((1 - sqrt_rho) * sqrt_rho < 0, new_sqrt_rho, sqrt_rho)
    rho = srf ** 2

    w_e = (u ** 2 + 2.0) * (x0c ** 2 + x1c ** 2) - 2.0 * _DIM
    pos = rho > 0
    csq = jnp.sqrt(jnp.where(pos, rho, 1.0)) * pos
    xn0 = x0c + drift0 * rho + diff0 * csq
    xn1 = x1c + drift1 * rho + diff1 * csq
    disc_n = disce * jnp.exp(-_GAMMA * _DT * rho)
    y_exit = (y + rho * w_e * _DT * disce) + disc_n * (xn0 ** 2 + xn1 ** 2)
    y_active = y + disc * (x0c ** 2 + x1c ** 2)
    y_ref[...] = jnp.where(flag > 0.0, y_active, y_exit)


def kernel(x0, dw, u):
    num_sample, dim = x0.shape
    xT = x0.T.reshape(dim, _ROWS, _LANES)
    dwT = dw.transpose(2, 1, 0).reshape(_N, dim, _ROWS, _LANES)
    u1 = jnp.reshape(u, (1,)).astype(jnp.float32)
    y = pl.pallas_call(
        _scan_kernel,
        grid=(_ROWS // _BS,),
        in_specs=[
            pl.BlockSpec(memory_space=pltpu.SMEM),
            pl.BlockSpec((dim, _BS, _LANES), lambda i: (0, i, 0)),
            pl.BlockSpec((_N, dim, _BS, _LANES), lambda i: (0, 0, i, 0)),
        ],
        out_specs=pl.BlockSpec((_BS, _LANES), lambda i: (i, 0)),
        out_shape=jax.ShapeDtypeStruct((_ROWS, _LANES), jnp.float32),
    )(u1, xT, dwT)
    return y.reshape(num_sample, 1)
